# Initial kernel scaffold; baseline (speedup 1.0000x reference)
import jax
import jax.numpy as jnp
from jax.experimental import pallas as pl


def kernel(x, W_router, W_gate, W_up, W_down):
    raise NotImplementedError("write your pallas kernel here")


# speedup vs baseline: 1.4085x; 1.4085x over previous
import functools

import jax
import jax.numpy as jnp
from jax.experimental import pallas as pl
from jax.experimental.pallas import tpu as pltpu


def _router_body(x_ref, wr_ref, coef_ref, *, E):
    x = x_ref[...]
    wr = wr_ref[...]
    logits = jax.lax.dot_general(
        x, wr, (((1,), (1,)), ((), ())), preferred_element_type=jnp.float32
    )
    T = logits.shape[0]
    eio = jax.lax.broadcasted_iota(jnp.int32, (T, E), 1)
    m1 = jnp.max(logits, axis=-1, keepdims=True)
    idx1 = jnp.min(jnp.where(logits == m1, eio, E), axis=-1, keepdims=True)
    masked = jnp.where(eio == idx1, -1e30, logits)
    m2 = jnp.max(masked, axis=-1, keepdims=True)
    idx2 = jnp.min(jnp.where(masked == m2, eio, E), axis=-1, keepdims=True)
    e2 = jnp.exp(m2 - m1)
    w1 = 1.0 / (1.0 + e2)
    w2 = e2 / (1.0 + e2)
    coef = jnp.where(eio == idx1, w1, 0.0) + jnp.where(eio == idx2, w2, 0.0)
    coef_ref[...] = coef


def _ffn_body(x_ref, wg_ref, wu_ref, wd_ref, coef_ref, out_ref, acc_ref, *, NF):
    e = pl.program_id(0)
    f = pl.program_id(1)
    E = coef_ref.shape[1]

    @pl.when((e == 0) & (f == 0))
    def _():
        out_ref[...] = jnp.zeros_like(out_ref)

    x = x_ref[...]
    wg = wg_ref[0]
    wu = wu_ref[0]
    xg = jax.lax.dot_general(
        x, wg, (((1,), (1,)), ((), ())), preferred_element_type=jnp.float32
    )
    xu = jax.lax.dot_general(
        x, wu, (((1,), (1,)), ((), ())), preferred_element_type=jnp.float32
    )
    h = (xg / (1.0 + jnp.exp(-xg))) * xu
    wd = wd_ref[0]
    part = jax.lax.dot_general(
        h, wd, (((1,), (1,)), ((), ())), preferred_element_type=jnp.float32
    )

    @pl.when(f == 0)
    def _():
        acc_ref[...] = part

    @pl.when(f > 0)
    def _():
        acc_ref[...] = acc_ref[...] + part

    @pl.when(f == NF - 1)
    def _():
        onehot = (jax.lax.broadcasted_iota(jnp.int32, (E, 1), 0) == e).astype(
            jnp.float32
        )
        coefcol = jax.lax.dot_general(
            coef_ref[...], onehot, (((1,), (0,)), ((), ())),
            preferred_element_type=jnp.float32,
        )
        out_ref[...] = out_ref[...] + acc_ref[...] * coefcol


def kernel(x, W_router, W_gate, W_up, W_down):
    T, D = x.shape
    E, F, _ = W_gate.shape
    FT = min(512, F)
    NF = F // FT

    coef = pl.pallas_call(
        functools.partial(_router_body, E=E),
        out_shape=jax.ShapeDtypeStruct((T, E), jnp.float32),
    )(x, W_router)

    out = pl.pallas_call(
        functools.partial(_ffn_body, NF=NF),
        grid=(E, NF),
        in_specs=[
            pl.BlockSpec((T, D), lambda e, f: (0, 0)),
            pl.BlockSpec((1, FT, D), lambda e, f: (e, f, 0)),
            pl.BlockSpec((1, FT, D), lambda e, f: (e, f, 0)),
            pl.BlockSpec((1, D, FT), lambda e, f: (e, 0, f)),
            pl.BlockSpec((T, E), lambda e, f: (0, 0)),
        ],
        out_specs=pl.BlockSpec((T, D), lambda e, f: (0, 0)),
        scratch_shapes=[pltpu.VMEM((T, D), jnp.float32)],
        out_shape=jax.ShapeDtypeStruct((T, D), jnp.float32),
    )(x, W_gate, W_up, W_down, coef)
    return out


# speedup vs baseline: 1.4273x; 1.0134x over previous
import functools

import jax
import jax.numpy as jnp
from jax import lax
from jax.experimental import pallas as pl
from jax.experimental.pallas import tpu as pltpu
from jax.experimental.pallas import tpu_sc as plsc

BLK = 256
MAXB = 24
PADROWS = MAXB * BLK
NW = 32
ACHUNK = 4
CW = 32
LANES = 16
WPAD = 128


def _route_body(x_ref, wr_ref, tril_ref, dest_ref, w_ref, blk_ref):
    T = x_ref.shape[0]
    E = wr_ref.shape[0]
    x = x_ref[...]
    logits = lax.dot_general(
        x, wr_ref[...], (((1,), (1,)), ((), ())),
        preferred_element_type=jnp.float32)
    eio = lax.broadcasted_iota(jnp.int32, (T, E), 1)
    m1 = jnp.max(logits, axis=-1, keepdims=True)
    idx1 = jnp.min(jnp.where(logits == m1, eio, E), axis=-1, keepdims=True)
    masked = jnp.where(eio == idx1, -1e30, logits)
    m2 = jnp.max(masked, axis=-1, keepdims=True)
    idx2 = jnp.min(jnp.where(masked == m2, eio, E), axis=-1, keepdims=True)
    e2 = jnp.exp(m2 - m1)
    w1 = 1.0 / (1.0 + e2)
    w2 = e2 / (1.0 + e2)
    sel = ((eio == idx1) | (eio == idx2)).astype(jnp.float32)

    pos = lax.dot_general(
        tril_ref[...], sel, (((1,), (0,)), ((), ())),
        preferred_element_type=jnp.float32)
    ones_row = jnp.ones((1, T), jnp.float32)
    counts = lax.dot_general(
        ones_row, sel, (((1,), (0,)), ((), ())),
        preferred_element_type=jnp.float32)
    padded = jnp.ceil(counts / BLK) * BLK
    r8 = lax.broadcasted_iota(jnp.int32, (E, E), 0)
    c8 = lax.broadcasted_iota(jnp.int32, (E, E), 1)
    upper = (r8 < c8).astype(jnp.float32)
    offs = lax.dot_general(
        padded, upper, (((1,), (0,)), ((), ())),
        preferred_element_type=jnp.float32)
    basep = offs + pos
    d1 = jnp.sum(jnp.where(eio == idx1, basep, 0.0), axis=-1, keepdims=True)
    d2 = jnp.sum(jnp.where(eio == idx2, basep, 0.0), axis=-1, keepdims=True)
    dest_ref[...] = jnp.concatenate([d1, d2], axis=1).astype(jnp.int32)
    w_ref[...] = jnp.concatenate([w1, w2], axis=1)

    ones_col = jnp.ones((T, 1), jnp.float32)
    counts_c = lax.dot_general(
        sel, ones_col, (((0,), (0,)), ((), ())),
        preferred_element_type=jnp.float32)
    padded_c = jnp.ceil(counts_c / BLK) * BLK
    lower = (c8 < r8).astype(jnp.float32)
    offs_c = lax.dot_general(
        lower, padded_c, (((1,), (0,)), ((), ())),
        preferred_element_type=jnp.float32)
    starts = (offs_c / BLK).astype(jnp.int32)
    ends = ((offs_c + padded_c) / BLK).astype(jnp.int32)
    NBPAD = blk_ref.shape[1]
    bio = lax.broadcasted_iota(jnp.int32, (E, NBPAD), 1)
    inr = ((bio >= starts) & (bio < ends)).astype(jnp.float32)
    ecol = lax.broadcasted_iota(jnp.int32, (E, NBPAD), 0).astype(jnp.float32)
    blk_e = jnp.sum(inr * ecol, axis=0, keepdims=True)
    valid = jnp.sum(inr, axis=0, keepdims=True)
    blk_ref[...] = (blk_e + E * (1.0 - valid)).astype(jnp.int32)


def _route(x, W_router, tril):
    T = x.shape[0]
    return pl.pallas_call(
        _route_body,
        out_shape=[
            jax.ShapeDtypeStruct((T, 2), jnp.int32),
            jax.ShapeDtypeStruct((T, 2), jnp.float32),
            jax.ShapeDtypeStruct((1, 32), jnp.int32),
        ],
    )(x, W_router, tril)


def _ffn_body(s_ref, xs_ref, ws_ref, wg_ref, wu_ref, wd_ref, out_ref, *, E):
    b = pl.program_id(0)
    e = s_ref[b]

    @pl.when(e < E)
    def _():
        xb = xs_ref[...].astype(jnp.bfloat16)
        xg = lax.dot_general(
            xb, wg_ref[0], (((1,), (1,)), ((), ())),
            preferred_element_type=jnp.float32)
        xu = lax.dot_general(
            xb, wu_ref[0], (((1,), (1,)), ((), ())),
            preferred_element_type=jnp.float32)
        h = ((xg / (1.0 + jnp.exp(-xg))) * xu).astype(jnp.bfloat16)
        part = lax.dot_general(
            h, wd_ref[0], (((1,), (1,)), ((), ())),
            preferred_element_type=jnp.float32)
        out_ref[...] = part * ws_ref[...][:, 0:1]


def _ffn(blk, xs, ws, Wg, Wu, Wd):
    E, F, D = Wg.shape
    grid_spec = pltpu.PrefetchScalarGridSpec(
        num_scalar_prefetch=1,
        grid=(MAXB,),
        in_specs=[
            pl.BlockSpec((BLK, D), lambda b, s: (b, 0)),
            pl.BlockSpec((BLK, WPAD), lambda b, s: (b, 0)),
            pl.BlockSpec((1, F, D), lambda b, s: (jnp.minimum(s[b], E - 1), 0, 0)),
            pl.BlockSpec((1, F, D), lambda b, s: (jnp.minimum(s[b], E - 1), 0, 0)),
            pl.BlockSpec((1, D, F), lambda b, s: (jnp.minimum(s[b], E - 1), 0, 0)),
        ],
        out_specs=pl.BlockSpec((BLK, D), lambda b, s: (b, 0)),
    )
    return pl.pallas_call(
        functools.partial(_ffn_body, E=E),
        grid_spec=grid_spec,
        out_shape=jax.ShapeDtypeStruct((PADROWS, D), jnp.float32),
    )(blk, xs, ws, Wg, Wu, Wd)


def _dispatch_sc(x, tok3, dest3, w16):
    T, D = x.shape
    APW = ACHUNK * CW
    info = plsc.get_sparse_core_info()
    NC = info.num_cores
    mesh = plsc.VectorSubcoreMesh(core_axis_name="c", subcore_axis_name="s")

    @functools.partial(
        pl.kernel, mesh=mesh,
        out_type=[
            jax.ShapeDtypeStruct((PADROWS, D), jnp.float32),
            jax.ShapeDtypeStruct((PADROWS, WPAD), jnp.float32),
        ],
        scratch_types=[
            pltpu.VMEM((ACHUNK, CW), jnp.int32),
            pltpu.VMEM((ACHUNK, CW), jnp.int32),
            pltpu.VMEM((CW, D), jnp.float32),
            pltpu.VMEM((CW, WPAD), jnp.float32),
            pltpu.SemaphoreType.DMA,
        ],
    )
    def k(x_hbm, tok_hbm, dest_hbm, w16_hbm, xs_hbm, ws_hbm,
          tok_v, dest_v, buf, wbuf, sem):
        wid = lax.axis_index("s") * NC + lax.axis_index("c")
        pltpu.sync_copy(tok_hbm.at[wid], tok_v)
        pltpu.sync_copy(dest_hbm.at[wid], dest_v)
        for ci in range(ACHUNK):
            pltpu.sync_copy(w16_hbm.at[pl.ds(wid * APW + ci * CW, CW)], wbuf)
            pltpu.async_copy(x_hbm.at[tok_v.at[ci]], buf, sem).wait()
            pltpu.async_copy(buf, xs_hbm.at[dest_v.at[ci]], sem).wait()
            pltpu.async_copy(wbuf, ws_hbm.at[dest_v.at[ci]], sem).wait()

    return k(x, tok3, dest3, w16)


def _combine_sc(ye, dest3, T):
    D = ye.shape[1]
    TOKC = CW // 2
    info = plsc.get_sparse_core_info()
    NC = info.num_cores
    mesh = plsc.VectorSubcoreMesh(core_axis_name="c", subcore_axis_name="s")

    @functools.partial(
        pl.kernel, mesh=mesh,
        out_type=jax.ShapeDtypeStruct((T, D), jnp.float32),
        scratch_types=[
            pltpu.VMEM((ACHUNK, CW), jnp.int32),
            pltpu.VMEM((CW, D), jnp.float32),
            pltpu.VMEM((TOKC, D), jnp.float32),
            pltpu.SemaphoreType.DMA,
        ],
    )
    def k(ye_hbm, dest_hbm, out_hbm, dest_v, buf, obuf, sem):
        wid = lax.axis_index("s") * NC + lax.axis_index("c")
        pltpu.sync_copy(dest_hbm.at[wid], dest_v)
        for ci in range(ACHUNK):
            pltpu.async_copy(ye_hbm.at[dest_v.at[ci]], buf, sem).wait()
            for j in range(TOKC):

                def sbody(si, _, j=j):
                    sl = pl.ds(si * LANES, LANES)
                    obuf[j, sl] = buf[2 * j, sl] + buf[2 * j + 1, sl]
                    return 0

                lax.fori_loop(0, D // LANES, sbody, 0)
            pltpu.sync_copy(
                obuf, out_hbm.at[pl.ds(wid * (ACHUNK * TOKC) + ci * TOKC, TOKC)])

    return k(ye, dest3)


def kernel(x, W_router, W_gate, W_up, W_down):
    T, D = x.shape
    E = W_gate.shape[0]
    tril = jnp.tril(jnp.ones((T, T), jnp.float32), -1)
    dest, w, blkmap = _route(x, W_router, tril)
    tok3 = (jnp.arange(T * 2, dtype=jnp.int32) // 2).reshape(NW, ACHUNK, CW)
    dest3 = dest.reshape(NW, ACHUNK, CW)
    w16 = jnp.broadcast_to(w.reshape(-1)[:, None], (T * 2, WPAD))
    xs, ws = _dispatch_sc(x, tok3, dest3, w16)
    ye = _ffn(blkmap.reshape(-1), xs, ws,
              W_gate.astype(jnp.bfloat16),
              W_up.astype(jnp.bfloat16),
              W_down.astype(jnp.bfloat16))
    return _combine_sc(ye, dest3, T)
